# baseline (speedup 1.0000x reference)
import functools

import jax
import jax.numpy as jnp
from jax import lax
from jax.experimental import pallas as pl
from jax.experimental.pallas import tpu as pltpu
from jax.experimental.pallas import tpu_sc as plsc

N = 8192
D = 768
E = 64
K = 8
BLK = 256
GRID = N // BLK

_NC = 2
_NS = 16
_NW = _NC * _NS
_BPW = (N * K) // _NW
_CH = 64
_NCH = _BPW // _CH


def _router_body(x_ref, w1_ref, b1_ref, w2_ref, b2_ref, keys_ref,
                 idx_ref, scr_ref, imp_ref, load_ref, loss_ref, kn_ref):
    i = pl.program_id(0)

    @pl.when(i == 0)
    def _init():
        imp_ref[...] = jnp.zeros_like(imp_ref)
        load_ref[...] = jnp.zeros_like(load_ref)

    keys = keys_ref[...]
    kn = keys / jnp.maximum(
        jnp.sqrt(jnp.sum(keys * keys, axis=1, keepdims=True)), 1e-12)

    kn_ref[...] = kn

    x = x_ref[...]
    h = lax.dot_general(x, w1_ref[...], (((1,), (1,)), ((), ())),
                        preferred_element_type=jnp.float32)
    h = jnp.maximum(h + b1_ref[...], 0.0)
    q = lax.dot_general(h, w2_ref[...], (((1,), (1,)), ((), ())),
                        preferred_element_type=jnp.float32)
    q = q + b2_ref[...]
    qn = q / jnp.maximum(jnp.sqrt(jnp.sum(q * q, axis=1, keepdims=True)),
                         1e-12)

    logits_t = lax.dot_general(kn, qn, (((1,), (1,)), ((), ())),
                               preferred_element_type=jnp.float32)

    eidx_t = lax.broadcasted_iota(jnp.int32, (E, BLK), 0)
    work = logits_t
    vals = []
    idxs = []
    for _ in range(K):
        m = jnp.max(work, axis=0, keepdims=True)
        a = jnp.min(jnp.where(work >= m, eidx_t, E), axis=0,
                    keepdims=True)
        vals.append(m)
        idxs.append(a)
        work = jnp.where(eidx_t == a, -jnp.inf, work)

    tv_t = jnp.concatenate(vals, axis=0)
    ti_t = jnp.concatenate(idxs, axis=0)

    ex = jnp.exp(tv_t - vals[0])
    scores_t = ex / jnp.sum(ex, axis=0, keepdims=True)

    eye_k = (lax.broadcasted_iota(jnp.int32, (K, K), 0) ==
             lax.broadcasted_iota(jnp.int32, (K, K), 1)).astype(jnp.float32)
    scores = lax.dot_general(scores_t, eye_k, (((0,), (0,)), ((), ())),
                             preferred_element_type=jnp.float32)
    idx_f = lax.dot_general(ti_t.astype(jnp.float32), eye_k,
                            (((0,), (0,)), ((), ())),
                            preferred_element_type=jnp.float32)
    idx_ref[...] = idx_f.astype(jnp.int32)
    scr_ref[...] = scores

    scf = jnp.zeros((E, BLK), jnp.float32)
    for j in range(K):
        scf = scf + jnp.where(eidx_t == idxs[j], scores_t[j:j + 1, :], 0.0)

    imp_ref[...] += jnp.sum(scf, axis=1, keepdims=True)
    load_ref[...] += jnp.sum((scf > 0).astype(jnp.int32), axis=1,
                             keepdims=True)

    @pl.when(i == GRID - 1)
    def _loss():
        def cv2(v):
            mean = jnp.sum(v) / E
            var = jnp.sum((v - mean) ** 2) / (E - 1)
            return var / (mean * mean + 1e-10)
        impf = imp_ref[...]
        loadf = load_ref[...].astype(jnp.float32)
        loss_ref[0, 0] = 0.01 * (cv2(impf) + cv2(loadf))


_NB = 2


def _sc_gather_body(kn_hbm, idx_hbm, out_hbm, idx_v,
                    r0, r1, g0, g1, w0, w1):
    rows = (r0, r1)
    gs = (g0, g1)
    ws = (w0, w1)
    wid = lax.axis_index("s") * _NC + lax.axis_index("c")
    base = wid * _BPW
    pltpu.sync_copy(idx_hbm.at[pl.ds(base, _BPW)], idx_v)

    def start_gather(c, r):
        pltpu.async_copy(kn_hbm.at[idx_v.at[pl.ds(c * _CH, _CH)]],
                         rows[r], gs[r])

    def drain_gather(r):
        pltpu.make_async_copy(kn_hbm.at[pl.ds(0, _CH)], rows[r],
                              gs[r]).wait()

    def start_write(c, r):
        pltpu.async_copy(rows[r], out_hbm.at[pl.ds(base + c * _CH, _CH)],
                         ws[r])

    def drain_write(c, r):
        pltpu.make_async_copy(rows[r],
                              out_hbm.at[pl.ds(base + c * _CH, _CH)],
                              ws[r]).wait()

    for r in range(_NB):
        start_gather(r, r)

    def ring(t, carry):
        for r in range(_NB):
            c = _NB * t + r
            drain_gather(r)
            start_write(c, r)
            nc = c + _NB

            @pl.when(nc < _NCH)
            def _():
                drain_write(c, r)
                start_gather(nc, r)
        return carry

    lax.fori_loop(0, _NCH // _NB, ring, 0)
    for r in range(_NB):
        drain_write(_NCH - _NB + r, r)


@jax.jit
def _router(x, W1, b1, W2, b2, keys):
    out = pl.pallas_call(
        _router_body,
        grid=(GRID,),
        in_specs=[
            pl.BlockSpec((BLK, D), lambda i: (i, 0)),
            pl.BlockSpec((D, D), lambda i: (0, 0)),
            pl.BlockSpec((1, D), lambda i: (0, 0)),
            pl.BlockSpec((D, D), lambda i: (0, 0)),
            pl.BlockSpec((1, D), lambda i: (0, 0)),
            pl.BlockSpec((E, D), lambda i: (0, 0)),
        ],
        out_specs=[
            pl.BlockSpec((BLK, K), lambda i: (i, 0)),
            pl.BlockSpec((BLK, K), lambda i: (i, 0)),
            pl.BlockSpec((E, 1), lambda i: (0, 0)),
            pl.BlockSpec((E, 1), lambda i: (0, 0)),
            pl.BlockSpec(memory_space=pltpu.SMEM),
            pl.BlockSpec((E, D), lambda i: (0, 0)),
        ],
        out_shape=[
            jax.ShapeDtypeStruct((N, K), jnp.int32),
            jax.ShapeDtypeStruct((N, K), jnp.float32),
            jax.ShapeDtypeStruct((E, 1), jnp.float32),
            jax.ShapeDtypeStruct((E, 1), jnp.int32),
            jax.ShapeDtypeStruct((1, 1), jnp.float32),
            jax.ShapeDtypeStruct((E, D), jnp.float32),
        ],
    )(x, W1, b1.reshape(1, D), W2, b2.reshape(1, D), keys)
    return out


@jax.jit
def _sc_gather(kn, idx_flat):
    mesh = plsc.VectorSubcoreMesh(core_axis_name="c", subcore_axis_name="s")
    run = functools.partial(
        pl.kernel,
        mesh=mesh,
        out_type=jax.ShapeDtypeStruct((N * K, D), jnp.float32),
        scratch_types=(
            [pltpu.VMEM((_BPW,), jnp.int32)]
            + [pltpu.VMEM((_CH, D), jnp.float32)] * _NB
            + [pltpu.SemaphoreType.DMA] * (2 * _NB)
        ),
    )(_sc_gather_body)
    return run(kn, idx_flat)


def kernel(x, W1, b1, W2, b2, keys):
    idx, scores, imp2, load2, loss2, kn = _router(x, W1, b1, W2, b2, keys)
    sel = _sc_gather(kn, idx.reshape(N * K)).reshape(N, K, D)
    return (idx, scores, loss2[0, 0], load2[:, 0], imp2[:, 0], sel)


# speedup vs baseline: 6.9645x; 6.9645x over previous
import functools

import jax
import jax.numpy as jnp
from jax import lax
from jax.experimental import pallas as pl
from jax.experimental.pallas import tpu as pltpu

N = 8192
D = 768
E = 64
K = 8
BLK = 512
GRID = N // BLK


def _router_body(x_ref, w1_ref, b1_ref, w2_ref, b2_ref, keys_ref,
                 idx_ref, scr_ref, imp_ref, load_ref, loss_ref, kn_ref,
                 sel_ref):
    i = pl.program_id(0)

    @pl.when(i == 0)
    def _init():
        imp_ref[...] = jnp.zeros_like(imp_ref)
        load_ref[...] = jnp.zeros_like(load_ref)

    keys = keys_ref[...]
    kn = keys / jnp.maximum(
        jnp.sqrt(jnp.sum(keys * keys, axis=1, keepdims=True)), 1e-12)

    @pl.when(i == 0)
    def _kn_out():
        kn_ref[...] = kn

    x = x_ref[...]
    h = lax.dot_general(x, w1_ref[...], (((1,), (1,)), ((), ())),
                        preferred_element_type=jnp.float32)
    h = jnp.maximum(h + b1_ref[...], 0.0)
    q = lax.dot_general(h, w2_ref[...], (((1,), (1,)), ((), ())),
                        preferred_element_type=jnp.float32)
    q = q + b2_ref[...]

    qn = q / jnp.maximum(jnp.sqrt(jnp.sum(q * q, axis=1, keepdims=True)),
                         1e-12)

    logits_t = lax.dot_general(kn, qn, (((1,), (1,)), ((), ())),
                               preferred_element_type=jnp.float32)

    eidx_t = lax.broadcasted_iota(jnp.int32, (E, BLK), 0)
    work = logits_t
    vals = []
    idxs = []
    for _ in range(K):
        m = jnp.max(work, axis=0, keepdims=True)
        a = jnp.min(jnp.where(work >= m, eidx_t, E), axis=0,
                    keepdims=True)
        vals.append(m)
        idxs.append(a)
        work = jnp.where(eidx_t == a, -jnp.inf, work)

    tv_t = jnp.concatenate(vals, axis=0)
    ti_t = jnp.concatenate(idxs, axis=0)

    ex = jnp.exp(tv_t - vals[0])
    scores_t = ex / jnp.sum(ex, axis=0, keepdims=True)

    eye_k = (lax.broadcasted_iota(jnp.int32, (K, K), 0) ==
             lax.broadcasted_iota(jnp.int32, (K, K), 1)).astype(jnp.float32)
    scores = lax.dot_general(scores_t, eye_k, (((0,), (0,)), ((), ())),
                             preferred_element_type=jnp.float32)
    idx_f = lax.dot_general(ti_t.astype(jnp.float32), eye_k,
                            (((0,), (0,)), ((), ())),
                            preferred_element_type=jnp.float32)
    idx_ref[...] = idx_f.astype(jnp.int32)
    scr_ref[...] = scores

    scf = jnp.zeros((E, BLK), jnp.float32)
    for j in range(K):
        scf = scf + jnp.where(eidx_t == idxs[j], scores_t[j:j + 1, :], 0.0)

    top_idx = idx_f.astype(jnp.int32)
    oh = (top_idx[:, :, None] ==
          lax.broadcasted_iota(jnp.int32, (BLK, K, E), 2)).astype(jnp.float32)
    sel = lax.dot_general(oh.reshape(BLK * K, E), kn,
                          (((1,), (0,)), ((), ())),
                          preferred_element_type=jnp.float32)
    sel_ref[...] = sel.reshape(BLK, K, D)

    imp_ref[...] += jnp.sum(scf, axis=1, keepdims=True)
    load_ref[...] += jnp.sum((scf > 0).astype(jnp.int32), axis=1,
                             keepdims=True)

    @pl.when(i == GRID - 1)
    def _loss():
        def cv2(v):
            mean = jnp.sum(v) / E
            var = jnp.sum((v - mean) ** 2) / (E - 1)
            return var / (mean * mean + 1e-10)
        impf = imp_ref[...]
        loadf = load_ref[...].astype(jnp.float32)
        loss_ref[0, 0] = 0.01 * (cv2(impf) + cv2(loadf))


@functools.partial(jax.jit)
def _router(x, W1, b1, W2, b2, keys):
    out = pl.pallas_call(
        _router_body,
        grid=(GRID,),
        in_specs=[
            pl.BlockSpec((BLK, D), lambda i: (i, 0)),
            pl.BlockSpec((D, D), lambda i: (0, 0)),
            pl.BlockSpec((1, D), lambda i: (0, 0)),
            pl.BlockSpec((D, D), lambda i: (0, 0)),
            pl.BlockSpec((1, D), lambda i: (0, 0)),
            pl.BlockSpec((E, D), lambda i: (0, 0)),
        ],
        out_specs=[
            pl.BlockSpec((BLK, K), lambda i: (i, 0)),
            pl.BlockSpec((BLK, K), lambda i: (i, 0)),
            pl.BlockSpec((E, 1), lambda i: (0, 0)),
            pl.BlockSpec((E, 1), lambda i: (0, 0)),
            pl.BlockSpec(memory_space=pltpu.SMEM),
            pl.BlockSpec((E, D), lambda i: (0, 0)),
            pl.BlockSpec((BLK, K, D), lambda i: (i, 0, 0)),
        ],
        out_shape=[
            jax.ShapeDtypeStruct((N, K), jnp.int32),
            jax.ShapeDtypeStruct((N, K), jnp.float32),
            jax.ShapeDtypeStruct((E, 1), jnp.float32),
            jax.ShapeDtypeStruct((E, 1), jnp.int32),
            jax.ShapeDtypeStruct((1, 1), jnp.float32),
            jax.ShapeDtypeStruct((E, D), jnp.float32),
            jax.ShapeDtypeStruct((N, K, D), jnp.float32),
        ],
    )(x, W1, b1.reshape(1, D), W2, b2.reshape(1, D), keys)
    return out


def kernel(x, W1, b1, W2, b2, keys):
    idx, scores, imp2, load2, loss2, kn, sel = _router(
        x, W1, b1, W2, b2, keys)
    return (idx, scores, loss2[0, 0], load2[:, 0], imp2[:, 0], sel)
